# baseline (speedup 1.0000x reference)
import functools

import jax
import jax.numpy as jnp
from jax import lax
from jax.experimental import pallas as pl
from jax.experimental.pallas import tpu as pltpu
from jax.experimental.pallas import tpu_sc as plsc

LANES = 16
CHUNK = 128
UNROLL = 16


@functools.lru_cache(maxsize=None)
def _build(B, D, n_ent, n_rel):
    info = plsc.get_sparse_core_info()
    NC, NS = info.num_cores, info.num_subcores
    NW = NC * NS
    assert B % (NW * CHUNK) == 0
    b_per_w = B // NW
    n_ch = b_per_w // CHUNK
    half = D // 2
    assert half % LANES == 0
    n_vr = half // LANES

    mesh = plsc.VectorSubcoreMesh(core_axis_name="c", subcore_axis_name="s")

    @functools.partial(
        pl.kernel,
        mesh=mesh,
        out_type=jax.ShapeDtypeStruct((B,), jnp.float32),
        compiler_params=pltpu.CompilerParams(
            needs_layout_passes=False, use_tc_tiling_on_sc=False),
        scratch_types=[
            pltpu.VMEM((b_per_w,), jnp.int32),
            pltpu.VMEM((b_per_w,), jnp.int32),
            pltpu.VMEM((b_per_w,), jnp.int32),
            pltpu.VMEM((b_per_w, D), jnp.float32),
            pltpu.VMEM((b_per_w, D), jnp.float32),
            pltpu.VMEM((b_per_w, D), jnp.float32),
            pltpu.VMEM((b_per_w,), jnp.float32),
            pltpu.VMEM((LANES, LANES), jnp.float32),
        ]
        + [pltpu.SemaphoreType.DMA] * n_ch,
    )
    def compl_ex(s_idx_hbm, p_idx_hbm, o_idx_hbm, so_hbm, p_hbm, out_hbm,
                 sidx_v, pidx_v, oidx_v, s_v, p_v, o_v, out_v, m_v, *sems):
        wid = lax.axis_index("s") * NC + lax.axis_index("c")
        base = wid * b_per_w

        pltpu.sync_copy(s_idx_hbm.at[pl.ds(base, b_per_w)], sidx_v)
        pltpu.sync_copy(p_idx_hbm.at[pl.ds(base, b_per_w)], pidx_v)
        pltpu.sync_copy(o_idx_hbm.at[pl.ds(base, b_per_w)], oidx_v)

        copies = []
        for j in range(n_ch):
            sl = pl.ds(j * CHUNK, CHUNK)
            copies.append((
                pltpu.async_copy(so_hbm.at[sidx_v.at[sl]], s_v.at[sl], sems[j]),
                pltpu.async_copy(p_hbm.at[pidx_v.at[sl]], p_v.at[sl], sems[j]),
                pltpu.async_copy(so_hbm.at[oidx_v.at[sl]], o_v.at[sl], sems[j]),
            ))

        def partials_one(e):
            acc = None
            for v in range(n_vr):
                re_sl = pl.ds(v * LANES, LANES)
                im_sl = pl.ds(half + v * LANES, LANES)
                sr, si = s_v[e, re_sl], s_v[e, im_sl]
                pr, pi = p_v[e, re_sl], p_v[e, im_sl]
                orr, oi = o_v[e, re_sl], o_v[e, im_sl]
                t = pr * (sr * orr + si * oi) + pi * (sr * oi - si * orr)
                acc = t if acc is None else acc + t
            return acc

        row_iota = lax.iota(jnp.int32, LANES)

        for j in range(n_ch):
            for c in copies[j]:
                c.wait()

            def body(g, carry, j=j):
                e0 = j * CHUNK + g * UNROLL
                for t in range(UNROLL):
                    m_v[t, :] = partials_one(e0 + t)
                cols = [
                    plsc.load_gather(
                        m_v,
                        [row_iota, jnp.broadcast_to(jnp.int32(c), (LANES,))])
                    for c in range(LANES)
                ]
                while len(cols) > 1:
                    cols = [cols[i] + cols[i + 1]
                            for i in range(0, len(cols), 2)]
                out_v[pl.ds(e0, LANES)] = cols[0]
                return carry

            lax.fori_loop(0, CHUNK // UNROLL, body, 0)

        pltpu.sync_copy(out_v, out_hbm.at[pl.ds(base, b_per_w)])

    return compl_ex


def kernel(s_idx, p_idx, o_idx, emb_so, emb_p):
    B = s_idx.shape[0]
    D = emb_so.shape[1]
    fn = _build(B, D, emb_so.shape[0], emb_p.shape[0])
    so_lin = lax.optimization_barrier(emb_so.reshape(-1))
    return fn(
        s_idx.astype(jnp.int32),
        p_idx.astype(jnp.int32),
        o_idx.astype(jnp.int32),
        so_lin.reshape(emb_so.shape),
        emb_p,
    )
